# baseline (device time: 16516 ns/iter reference)
import jax
import jax.numpy as jnp
from jax import lax
from jax.experimental import pallas as pl
from jax.experimental.pallas import tpu as pltpu

N_DEV = 4
N_TOK = 1024
D_MODEL = 256
D_HID = 512
N_EXP = 16
E_LOCAL = N_EXP // N_DEV
M_PER = N_TOK // N_DEV
CAP = 160


def kernel(x, router_W, route_idx, expert_W):
    def body(x_hbm, rw_ref, idx_ref, ew_hbm, out_ref,
             x_ref, ew_ref, send_ref, recv_ref,
             load_sems, send_sems, recv_sems):
        my_pos = lax.axis_index("i")

        cp_x = pltpu.make_async_copy(x_hbm, x_ref, load_sems.at[0])
        cp_ew = pltpu.make_async_copy(ew_hbm, ew_ref, load_sems.at[1])
        cp_x.start()
        cp_ew.start()

        barrier_sem = pltpu.get_barrier_semaphore()
        for r in range(1, N_DEV):
            pl.semaphore_signal(
                barrier_sem, inc=1,
                device_id=(lax.rem(my_pos + r, N_DEV),),
                device_id_type=pl.DeviceIdType.MESH,
            )
        tril = (lax.broadcasted_iota(jnp.int32, (M_PER, M_PER), 1)
                < lax.broadcasted_iota(jnp.int32, (M_PER, M_PER), 0)
                ).astype(jnp.float32)
        slot_iota = lax.broadcasted_iota(jnp.int32, (M_PER, CAP), 1)

        def routed_mask(rs, dev):
            idx = idx_ref[pl.ds(rs, M_PER), :]
            lo = dev * E_LOCAL
            in0 = (idx[:, 0:1] >= lo) & (idx[:, 0:1] < lo + E_LOCAL)
            in1 = (idx[:, 1:2] >= lo) & (idx[:, 1:2] < lo + E_LOCAL)
            return (in0 | in1).astype(jnp.float32)

        def pack_matrix(rs, dev):
            mask = routed_mask(rs, dev)
            pos = jnp.dot(tril, mask, preferred_element_type=jnp.float32,
                          precision=lax.Precision.HIGHEST)
            sel = (slot_iota == pos.astype(jnp.int32)) & (mask > 0.5)
            return sel.astype(jnp.bfloat16)

        send_sels = [pack_matrix(lax.rem(my_pos + r, N_DEV) * M_PER, my_pos)
                     for r in range(1, N_DEV)]
        recv_sels = [pack_matrix(my_pos * M_PER,
                                 lax.rem(my_pos + 2 * N_DEV - r, N_DEV))
                     for r in range(1, N_DEV)]

        cp_x.wait()
        cp_ew.wait()

        eww = ew_ref[:, :, :].astype(jnp.bfloat16).reshape(
            E_LOCAL * D_MODEL, D_HID)

        def tile_partial(rs, m):
            xb = x_ref[pl.ds(rs, m), :]
            scores = jnp.dot(xb, rw_ref[:, :],
                             preferred_element_type=jnp.float32,
                             precision=lax.Precision.HIGHEST)
            s_max = jnp.max(scores, axis=-1, keepdims=True)
            e = jnp.exp(scores - s_max)
            probs = e / jnp.sum(e, axis=-1, keepdims=True)
            idx = idx_ref[pl.ds(rs, m), :]
            t_iota = lax.broadcasted_iota(jnp.int32, (m, N_EXP), 1)
            hit0 = idx[:, 0:1] == t_iota
            hit1 = idx[:, 1:2] == t_iota
            g0 = jnp.sum(jnp.where(hit0, probs, 0.0), axis=-1, keepdims=True)
            g1 = jnp.sum(jnp.where(hit1, probs, 0.0), axis=-1, keepdims=True)
            wb = jnp.where(hit0 | hit1, probs, 0.0) / (g0 + g1)
            xb16 = xb.astype(jnp.bfloat16)
            parts = []
            for j in range(E_LOCAL):
                ge = my_pos * E_LOCAL + j
                coeff = jnp.sum(jnp.where(t_iota == ge, wb, 0.0),
                                axis=-1, keepdims=True)
                parts.append(xb16 * coeff.astype(jnp.bfloat16))
            return jnp.dot(jnp.concatenate(parts, axis=1), eww,
                           preferred_element_type=jnp.float32)

        rdmas = []
        for r in range(1, N_DEV):
            dst = lax.rem(my_pos + r, N_DEV)
            partial = tile_partial(dst * M_PER, M_PER).astype(jnp.bfloat16)
            sel = send_sels[r - 1]
            send_ref[r - 1, :, :] = lax.dot_general(
                sel, partial, (((0,), (0,)), ((), ())),
                preferred_element_type=jnp.float32).astype(jnp.bfloat16)
            if r == 1:
                pl.semaphore_wait(barrier_sem, N_DEV - 1)
            rdma = pltpu.make_async_remote_copy(
                src_ref=send_ref.at[r - 1],
                dst_ref=recv_ref.at[r - 1],
                send_sem=send_sems.at[r - 1],
                recv_sem=recv_sems.at[r - 1],
                device_id=(dst,),
                device_id_type=pl.DeviceIdType.MESH,
            )
            rdma.start()
            rdmas.append(rdma)

        total = tile_partial(my_pos * M_PER, M_PER)
        for r in range(1, N_DEV):
            rdmas[r - 1].wait_recv()
            total = total + jnp.dot(recv_sels[r - 1], recv_ref[r - 1, :, :],
                                    preferred_element_type=jnp.float32)
        out_ref[:, :] = total

        for rdma in rdmas:
            rdma.wait_send()

    return pl.pallas_call(
        body,
        out_shape=jax.ShapeDtypeStruct((M_PER, D_HID), jnp.float32),
        in_specs=[
            pl.BlockSpec(memory_space=pl.ANY),
            pl.BlockSpec(memory_space=pltpu.VMEM),
            pl.BlockSpec(memory_space=pltpu.VMEM),
            pl.BlockSpec(memory_space=pl.ANY),
        ],
        out_specs=pl.BlockSpec(memory_space=pltpu.VMEM),
        scratch_shapes=[
            pltpu.VMEM((N_TOK, D_MODEL), jnp.float32),
            pltpu.VMEM((E_LOCAL, D_MODEL, D_HID), jnp.float32),
            pltpu.VMEM((N_DEV - 1, CAP, D_HID), jnp.bfloat16),
            pltpu.VMEM((N_DEV - 1, CAP, D_HID), jnp.bfloat16),
            pltpu.SemaphoreType.DMA((2,)),
            pltpu.SemaphoreType.DMA((N_DEV - 1,)),
            pltpu.SemaphoreType.DMA((N_DEV - 1,)),
        ],
        compiler_params=pltpu.CompilerParams(collective_id=0),
    )(
        pltpu.with_memory_space_constraint(x, pltpu.MemorySpace.HBM),
        router_W,
        route_idx,
        pltpu.with_memory_space_constraint(expert_W, pltpu.MemorySpace.HBM),
    )


# device time: 16505 ns/iter; 1.0007x vs baseline; 1.0007x over previous
import jax
import jax.numpy as jnp
from jax import lax
from jax.experimental import pallas as pl
from jax.experimental.pallas import tpu as pltpu

N_DEV = 4
N_TOK = 1024
D_MODEL = 256
D_HID = 512
N_EXP = 16
E_LOCAL = N_EXP // N_DEV
M_PER = N_TOK // N_DEV
CAP = 160


def kernel(x, router_W, route_idx, expert_W):
    def body(x_hbm, rw_ref, idx_ref, ew_hbm, out_ref,
             x_ref, ew_ref, send_ref, recv_ref,
             load_sems, send_sems, recv_sems):
        my_pos = lax.axis_index("i")

        cp_x = pltpu.make_async_copy(x_hbm, x_ref, load_sems.at[0])
        cp_ew = pltpu.make_async_copy(ew_hbm, ew_ref, load_sems.at[1])
        cp_x.start()
        cp_ew.start()

        barrier_sem = pltpu.get_barrier_semaphore()
        for r in range(1, N_DEV):
            pl.semaphore_signal(
                barrier_sem, inc=1,
                device_id=(lax.rem(my_pos + r, N_DEV),),
                device_id_type=pl.DeviceIdType.MESH,
            )
        tril = (lax.broadcasted_iota(jnp.int32, (M_PER, M_PER), 1)
                < lax.broadcasted_iota(jnp.int32, (M_PER, M_PER), 0)
                ).astype(jnp.float32)
        slot_iota = lax.broadcasted_iota(jnp.int32, (M_PER, CAP), 1)

        def routed_mask(rs, dev):
            idx = idx_ref[pl.ds(rs, M_PER), :]
            lo = dev * E_LOCAL
            in0 = (idx[:, 0:1] >= lo) & (idx[:, 0:1] < lo + E_LOCAL)
            in1 = (idx[:, 1:2] >= lo) & (idx[:, 1:2] < lo + E_LOCAL)
            return (in0 | in1).astype(jnp.float32)

        def pack_matrix(rs, dev):
            mask = routed_mask(rs, dev)
            pos = jnp.dot(tril, mask, preferred_element_type=jnp.float32,
                          precision=lax.Precision.HIGHEST)
            sel = (slot_iota == pos.astype(jnp.int32)) & (mask > 0.5)
            return sel.astype(jnp.bfloat16)

        send_sels = [pack_matrix(lax.rem(my_pos + r, N_DEV) * M_PER, my_pos)
                     for r in range(1, N_DEV)]
        recv_sels = [pack_matrix(my_pos * M_PER,
                                 lax.rem(my_pos + 2 * N_DEV - r, N_DEV))
                     for r in range(1, N_DEV)]

        cp_x.wait()
        cp_ew.wait()

        eww = ew_ref[:, :, :].astype(jnp.bfloat16).reshape(
            E_LOCAL * D_MODEL, D_HID)

        def tile_partial(rs, m):
            xb = x_ref[pl.ds(rs, m), :]
            scores = jnp.dot(xb, rw_ref[:, :],
                             preferred_element_type=jnp.float32,
                             precision=lax.Precision.HIGHEST)
            s_max = jnp.max(scores, axis=-1, keepdims=True)
            e = jnp.exp(scores - s_max)
            probs = e / jnp.sum(e, axis=-1, keepdims=True)
            idx = idx_ref[pl.ds(rs, m), :]
            t_iota = lax.broadcasted_iota(jnp.int32, (m, N_EXP), 1)
            hit0 = idx[:, 0:1] == t_iota
            hit1 = idx[:, 1:2] == t_iota
            g0 = jnp.sum(jnp.where(hit0, probs, 0.0), axis=-1, keepdims=True)
            g1 = jnp.sum(jnp.where(hit1, probs, 0.0), axis=-1, keepdims=True)
            wb = jnp.where(hit0 | hit1, probs, 0.0) / (g0 + g1)
            parts = []
            for j in range(E_LOCAL):
                ge = my_pos * E_LOCAL + j
                coeff = jnp.sum(jnp.where(t_iota == ge, wb, 0.0),
                                axis=-1, keepdims=True)
                parts.append((xb * coeff).astype(jnp.bfloat16))
            return jnp.dot(jnp.concatenate(parts, axis=1), eww,
                           preferred_element_type=jnp.float32)

        rdmas = []
        for r in range(1, N_DEV):
            dst = lax.rem(my_pos + r, N_DEV)
            partial = tile_partial(dst * M_PER, M_PER).astype(jnp.bfloat16)
            sel = send_sels[r - 1]
            send_ref[r - 1, :, :] = lax.dot_general(
                sel, partial, (((0,), (0,)), ((), ())),
                preferred_element_type=jnp.float32).astype(jnp.bfloat16)
            if r == 1:
                pl.semaphore_wait(barrier_sem, N_DEV - 1)
            rdma = pltpu.make_async_remote_copy(
                src_ref=send_ref.at[r - 1],
                dst_ref=recv_ref.at[r - 1],
                send_sem=send_sems.at[r - 1],
                recv_sem=recv_sems.at[r - 1],
                device_id=(dst,),
                device_id_type=pl.DeviceIdType.MESH,
            )
            rdma.start()
            rdmas.append(rdma)

        total = tile_partial(my_pos * M_PER, M_PER)
        for r in range(1, N_DEV):
            rdmas[r - 1].wait_recv()
            total = total + jnp.dot(recv_sels[r - 1], recv_ref[r - 1, :, :],
                                    preferred_element_type=jnp.float32)
        out_ref[:, :] = total

        for rdma in rdmas:
            rdma.wait_send()

    return pl.pallas_call(
        body,
        out_shape=jax.ShapeDtypeStruct((M_PER, D_HID), jnp.float32),
        in_specs=[
            pl.BlockSpec(memory_space=pl.ANY),
            pl.BlockSpec(memory_space=pltpu.VMEM),
            pl.BlockSpec(memory_space=pltpu.VMEM),
            pl.BlockSpec(memory_space=pl.ANY),
        ],
        out_specs=pl.BlockSpec(memory_space=pltpu.VMEM),
        scratch_shapes=[
            pltpu.VMEM((N_TOK, D_MODEL), jnp.float32),
            pltpu.VMEM((E_LOCAL, D_MODEL, D_HID), jnp.float32),
            pltpu.VMEM((N_DEV - 1, CAP, D_HID), jnp.bfloat16),
            pltpu.VMEM((N_DEV - 1, CAP, D_HID), jnp.bfloat16),
            pltpu.SemaphoreType.DMA((2,)),
            pltpu.SemaphoreType.DMA((N_DEV - 1,)),
            pltpu.SemaphoreType.DMA((N_DEV - 1,)),
        ],
        compiler_params=pltpu.CompilerParams(collective_id=0),
    )(
        pltpu.with_memory_space_constraint(x, pltpu.MemorySpace.HBM),
        router_W,
        route_idx,
        pltpu.with_memory_space_constraint(expert_W, pltpu.MemorySpace.HBM),
    )
